# Initial kernel scaffold; baseline (speedup 1.0000x reference)
import functools

import jax
import jax.numpy as jnp
from jax import lax
from jax.experimental import pallas as pl
from jax.experimental.pallas import tpu as pltpu
from jax.experimental.pallas import tpu_sc as plsc

N = 10000
E = 320000
D = 128
DE = 16
Z2 = 2 * D

NC = 2
NS = 16
NW = NC * NS
EPW = E // NW
CHUNK = 80
ROWS_PER_TILE = N // NS

_mesh = plsc.VectorSubcoreMesh(core_axis_name="c", subcore_axis_name="s")


@functools.partial(
    pl.kernel,
    out_type=(
        jax.ShapeDtypeStruct((E, Z2), jnp.float32),
        jax.ShapeDtypeStruct((E, Z2), jnp.float32),
    ),
    mesh=_mesh,
    scratch_types=[
        pltpu.VMEM((CHUNK,), jnp.int32),
        pltpu.VMEM((CHUNK,), jnp.int32),
        pltpu.VMEM((CHUNK, Z2), jnp.float32),
        pltpu.VMEM((CHUNK, Z2), jnp.float32),
        pltpu.SemaphoreType.DMA,
        pltpu.SemaphoreType.DMA,
    ],
)
def _gather_call(td_hbm, ts_hbm, di_hbm, si_hbm, gd_hbm, gs_hbm,
                 idx_d, idx_s, rows_d, rows_s, sem_d, sem_s):
    wid = lax.axis_index("s") * NC + lax.axis_index("c")
    base = wid * EPW

    def body(k, carry):
        off = base + k * CHUNK
        pltpu.sync_copy(di_hbm.at[pl.ds(off, CHUNK)], idx_d)
        pltpu.sync_copy(si_hbm.at[pl.ds(off, CHUNK)], idx_s)
        cp_d = pltpu.async_copy(td_hbm.at[idx_d], rows_d, sem_d)
        cp_s = pltpu.async_copy(ts_hbm.at[idx_s], rows_s, sem_s)
        cp_d.wait()
        cp_s.wait()
        pltpu.sync_copy(rows_d, gd_hbm.at[pl.ds(off, CHUNK)])
        pltpu.sync_copy(rows_s, gs_hbm.at[pl.ds(off, CHUNK)])
        return carry

    lax.fori_loop(0, EPW // CHUNK, body, 0)


@functools.partial(
    pl.kernel,
    out_type=jax.ShapeDtypeStruct((NC, N, D), jnp.float32),
    mesh=_mesh,
    scratch_types=[
        pltpu.VMEM((CHUNK,), jnp.int32),
        pltpu.VMEM((CHUNK, D), jnp.float32),
        pltpu.VMEM_SHARED((N, D), jnp.float32),
    ],
)
def _scatter_call(msg_hbm, di_hbm, zeros_hbm, out_hbm, idx_v, rows_v, agg_sh):
    cid = lax.axis_index("c")
    sid = lax.axis_index("s")
    wid = sid * NC + cid

    @pl.when(sid == 0)
    def _():
        pltpu.sync_copy(zeros_hbm, agg_sh)

    plsc.subcore_barrier()

    base = wid * EPW

    def body(k, carry):
        off = base + k * CHUNK
        pltpu.sync_copy(di_hbm.at[pl.ds(off, CHUNK)], idx_v)
        pltpu.sync_copy(msg_hbm.at[pl.ds(off, CHUNK)], rows_v)
        pltpu.sync_copy(rows_v, agg_sh.at[idx_v], add=True)
        return carry

    lax.fori_loop(0, EPW // CHUNK, body, 0)
    plsc.subcore_barrier()

    def out_body(j, carry):
        r0 = sid * ROWS_PER_TILE + j * CHUNK
        pltpu.sync_copy(agg_sh.at[pl.ds(r0, CHUNK)], rows_v)
        pltpu.sync_copy(rows_v, out_hbm.at[cid, pl.ds(r0, CHUNK)])
        return carry

    lax.fori_loop(0, ROWS_PER_TILE // CHUNK, out_body, 0)
    rem = ROWS_PER_TILE % CHUNK
    r0 = sid * ROWS_PER_TILE + (ROWS_PER_TILE // CHUNK) * CHUNK
    pltpu.sync_copy(agg_sh.at[pl.ds(r0, rem)], rows_v.at[pl.ds(0, rem)])
    pltpu.sync_copy(rows_v.at[pl.ds(0, rem)], out_hbm.at[cid, pl.ds(r0, rem)])


def _tables_body(x_ref, w1_ref, w2_ref, td_ref, ts_ref):
    xb = x_ref[...]
    td_ref[...] = jnp.dot(xb, w1_ref[...], preferred_element_type=jnp.float32)
    ts_ref[...] = jnp.dot(xb, w2_ref[...], preferred_element_type=jnp.float32)


BN_TAB = 2000
_tables_call = pl.pallas_call(
    _tables_body,
    grid=(N // BN_TAB,),
    in_specs=[
        pl.BlockSpec((BN_TAB, D), lambda i: (i, 0)),
        pl.BlockSpec((D, Z2), lambda i: (0, 0)),
        pl.BlockSpec((D, Z2), lambda i: (0, 0)),
    ],
    out_specs=[
        pl.BlockSpec((BN_TAB, Z2), lambda i: (i, 0)),
        pl.BlockSpec((BN_TAB, Z2), lambda i: (i, 0)),
    ],
    out_shape=[jax.ShapeDtypeStruct((N, Z2), jnp.float32)] * 2,
)


def _act_body(gd_ref, gs_ref, ea_ref, wet_ref, bcat_ref, msg_ref):
    u = (gd_ref[...] + gs_ref[...]
         + jnp.dot(ea_ref[...], wet_ref[...], preferred_element_type=jnp.float32)
         + bcat_ref[...])
    gate = jax.nn.sigmoid(u[:, :D])
    filt = jax.nn.softplus(u[:, D:])
    msg_ref[...] = gate * filt


BE_ACT = 4000
_act_call = pl.pallas_call(
    _act_body,
    grid=(E // BE_ACT,),
    in_specs=[
        pl.BlockSpec((BE_ACT, Z2), lambda i: (i, 0)),
        pl.BlockSpec((BE_ACT, Z2), lambda i: (i, 0)),
        pl.BlockSpec((BE_ACT, DE), lambda i: (i, 0)),
        pl.BlockSpec((DE, Z2), lambda i: (0, 0)),
        pl.BlockSpec((1, Z2), lambda i: (0, 0)),
    ],
    out_specs=pl.BlockSpec((BE_ACT, D), lambda i: (i, 0)),
    out_shape=jax.ShapeDtypeStruct((E, D), jnp.float32),
)


def _final_body(x_ref, p_ref, wd_ref, bd_ref, out_ref, acc_ref):
    i = pl.program_id(0)

    @pl.when(i == 0)
    def _():
        acc_ref[...] = jnp.zeros_like(acc_ref)

    s = jnp.maximum(x_ref[...] + p_ref[0] + p_ref[1], 0.0)
    acc_ref[...] += jnp.sum(s, axis=0, keepdims=True)

    @pl.when(i == pl.num_programs(0) - 1)
    def _():
        out_ref[...] = jnp.sum(acc_ref[...] * wd_ref[...]) + bd_ref[...]


BN_FIN = 1000
_final_call = pl.pallas_call(
    _final_body,
    grid=(N // BN_FIN,),
    in_specs=[
        pl.BlockSpec((BN_FIN, D), lambda i: (i, 0)),
        pl.BlockSpec((NC, BN_FIN, D), lambda i: (0, i, 0)),
        pl.BlockSpec((1, D), lambda i: (0, 0)),
        pl.BlockSpec((1, D), lambda i: (0, 0)),
    ],
    out_specs=pl.BlockSpec((1, D), lambda i: (0, 0)),
    out_shape=jax.ShapeDtypeStruct((1, D), jnp.float32),
    scratch_shapes=[pltpu.VMEM((1, D), jnp.float32)],
)


def kernel(x, edge_index, edge_attr, Wf, bf, Ws, bs, Wd, bd):
    ei = edge_index.astype(jnp.int32)
    src = ei[0]
    dst = ei[1]
    w1 = jnp.concatenate([Wf[:, :D], Ws[:, :D]], axis=0).T
    w2 = jnp.concatenate([Wf[:, D:2 * D], Ws[:, D:2 * D]], axis=0).T
    wet = jnp.concatenate([Wf[:, 2 * D:], Ws[:, 2 * D:]], axis=0).T
    bcat = jnp.concatenate([bf, bs]).reshape(1, Z2)

    td, ts = _tables_call(x, w1, w2)
    gd, gs = _gather_call(td, ts, dst, src)
    msg = _act_call(gd, gs, edge_attr, wet, bcat)
    partials = _scatter_call(msg, dst, jnp.zeros((N, D), jnp.float32))
    outv = _final_call(x, partials, Wd, jnp.broadcast_to(bd, (1, D)))
    return outv[0, :1]


# speedup vs baseline: 2.4749x; 2.4749x over previous
import functools

import jax
import jax.numpy as jnp
from jax import lax
from jax.experimental import pallas as pl
from jax.experimental.pallas import tpu as pltpu
from jax.experimental.pallas import tpu_sc as plsc

N = 10000
E = 320000
D = 128
DE = 16
Z2 = 2 * D

NC = 2
NS = 16
NW = NC * NS
EPW = E // NW
CHUNK = 80
ROWS_PER_TILE = N // NS

_mesh = plsc.VectorSubcoreMesh(core_axis_name="c", subcore_axis_name="s")


@functools.partial(
    pl.kernel,
    out_type=(
        jax.ShapeDtypeStruct((E, Z2), jnp.float32),
        jax.ShapeDtypeStruct((E, Z2), jnp.float32),
    ),
    mesh=_mesh,
    scratch_types=[
        pltpu.VMEM((CHUNK,), jnp.int32),
        pltpu.VMEM((CHUNK,), jnp.int32),
        pltpu.VMEM((CHUNK, Z2), jnp.float32),
        pltpu.VMEM((CHUNK, Z2), jnp.float32),
        pltpu.SemaphoreType.DMA,
        pltpu.SemaphoreType.DMA,
    ],
)
def _gather_call(td_hbm, ts_hbm, di_hbm, si_hbm, gd_hbm, gs_hbm,
                 idx_d, idx_s, rows_d, rows_s, sem_d, sem_s):
    wid = lax.axis_index("s") * NC + lax.axis_index("c")
    base = wid * EPW

    def body(k, carry):
        off = base + k * CHUNK
        pltpu.sync_copy(di_hbm.at[pl.ds(off, CHUNK)], idx_d)
        pltpu.sync_copy(si_hbm.at[pl.ds(off, CHUNK)], idx_s)
        cp_d = pltpu.async_copy(td_hbm.at[idx_d], rows_d, sem_d)
        cp_s = pltpu.async_copy(ts_hbm.at[idx_s], rows_s, sem_s)
        cp_d.wait()
        cp_s.wait()
        pltpu.sync_copy(rows_d, gd_hbm.at[pl.ds(off, CHUNK)])
        pltpu.sync_copy(rows_s, gs_hbm.at[pl.ds(off, CHUNK)])
        return carry

    lax.fori_loop(0, EPW // CHUNK, body, 0)


@functools.partial(
    pl.kernel,
    out_type=jax.ShapeDtypeStruct((NC, N, D), jnp.float32),
    mesh=_mesh,
    scratch_types=[
        pltpu.VMEM((CHUNK,), jnp.int32),
        pltpu.VMEM((CHUNK, D), jnp.float32),
        pltpu.VMEM_SHARED((N, D), jnp.float32),
    ],
)
def _scatter_call(msg_hbm, di_hbm, zeros_hbm, out_hbm, idx_v, rows_v, agg_sh):
    cid = lax.axis_index("c")
    sid = lax.axis_index("s")
    wid = sid * NC + cid

    @pl.when(sid == 0)
    def _():
        pltpu.sync_copy(zeros_hbm, agg_sh)

    plsc.subcore_barrier()

    base = wid * EPW

    def body(k, carry):
        off = base + k * CHUNK
        pltpu.sync_copy(di_hbm.at[pl.ds(off, CHUNK)], idx_v)
        pltpu.sync_copy(msg_hbm.at[pl.ds(off, CHUNK)], rows_v)
        pltpu.sync_copy(rows_v, agg_sh.at[idx_v], add=True)
        return carry

    lax.fori_loop(0, EPW // CHUNK, body, 0)
    plsc.subcore_barrier()

    def out_body(j, carry):
        r0 = sid * 640 + j * CHUNK
        pltpu.sync_copy(agg_sh.at[pl.ds(r0, CHUNK)], rows_v)
        pltpu.sync_copy(rows_v, out_hbm.at[cid, pl.ds(r0, CHUNK)])
        return carry

    nchunks = jnp.where(sid == NS - 1, (N - 640 * (NS - 1)) // CHUNK, 640 // CHUNK)
    lax.fori_loop(0, nchunks, out_body, 0)


def _tables_body(x_ref, w1_ref, w2_ref, td_ref, ts_ref):
    xb = x_ref[...]
    td_ref[...] = jnp.dot(xb, w1_ref[...], preferred_element_type=jnp.float32)
    ts_ref[...] = jnp.dot(xb, w2_ref[...], preferred_element_type=jnp.float32)


BN_TAB = 2000
_tables_call = pl.pallas_call(
    _tables_body,
    grid=(N // BN_TAB,),
    in_specs=[
        pl.BlockSpec((BN_TAB, D), lambda i: (i, 0)),
        pl.BlockSpec((D, Z2), lambda i: (0, 0)),
        pl.BlockSpec((D, Z2), lambda i: (0, 0)),
    ],
    out_specs=[
        pl.BlockSpec((BN_TAB, Z2), lambda i: (i, 0)),
        pl.BlockSpec((BN_TAB, Z2), lambda i: (i, 0)),
    ],
    out_shape=[jax.ShapeDtypeStruct((N, Z2), jnp.float32)] * 2,
)


def _act_body(gd_ref, gs_ref, ea_ref, wet_ref, bcat_ref, msg_ref):
    u = (gd_ref[...] + gs_ref[...]
         + jnp.dot(ea_ref[...], wet_ref[...], preferred_element_type=jnp.float32)
         + bcat_ref[...])
    gate = jax.nn.sigmoid(u[:, :D])
    filt = jax.nn.softplus(u[:, D:])
    msg_ref[...] = gate * filt


BE_ACT = 4000
_act_call = pl.pallas_call(
    _act_body,
    grid=(E // BE_ACT,),
    in_specs=[
        pl.BlockSpec((BE_ACT, Z2), lambda i: (i, 0)),
        pl.BlockSpec((BE_ACT, Z2), lambda i: (i, 0)),
        pl.BlockSpec((BE_ACT, DE), lambda i: (i, 0)),
        pl.BlockSpec((DE, Z2), lambda i: (0, 0)),
        pl.BlockSpec((1, Z2), lambda i: (0, 0)),
    ],
    out_specs=pl.BlockSpec((BE_ACT, D), lambda i: (i, 0)),
    out_shape=jax.ShapeDtypeStruct((E, D), jnp.float32),
)


def _final_body(x_ref, p_ref, wd_ref, bd_ref, out_ref, acc_ref):
    i = pl.program_id(0)

    @pl.when(i == 0)
    def _():
        acc_ref[...] = jnp.zeros_like(acc_ref)

    s = jnp.maximum(x_ref[...] + p_ref[0] + p_ref[1], 0.0)
    acc_ref[...] += jnp.sum(s, axis=0, keepdims=True)

    @pl.when(i == pl.num_programs(0) - 1)
    def _():
        out_ref[...] = jnp.sum(acc_ref[...] * wd_ref[...]) + bd_ref[...]


BN_FIN = 1000
_final_call = pl.pallas_call(
    _final_body,
    grid=(N // BN_FIN,),
    in_specs=[
        pl.BlockSpec((BN_FIN, D), lambda i: (i, 0)),
        pl.BlockSpec((NC, BN_FIN, D), lambda i: (0, i, 0)),
        pl.BlockSpec((1, D), lambda i: (0, 0)),
        pl.BlockSpec((1, D), lambda i: (0, 0)),
    ],
    out_specs=pl.BlockSpec((1, D), lambda i: (0, 0)),
    out_shape=jax.ShapeDtypeStruct((1, D), jnp.float32),
    scratch_shapes=[pltpu.VMEM((1, D), jnp.float32)],
)


def kernel(x, edge_index, edge_attr, Wf, bf, Ws, bs, Wd, bd):
    ei = edge_index.astype(jnp.int32)
    src = ei[0]
    dst = ei[1]
    w1 = jnp.concatenate([Wf[:, :D], Ws[:, :D]], axis=0).T
    w2 = jnp.concatenate([Wf[:, D:2 * D], Ws[:, D:2 * D]], axis=0).T
    wet = jnp.concatenate([Wf[:, 2 * D:], Ws[:, 2 * D:]], axis=0).T
    bcat = jnp.concatenate([bf, bs]).reshape(1, Z2)

    td, ts = _tables_call(x, w1, w2)
    gd, gs = _gather_call(td, ts, dst, src)
    msg = _act_call(gd, gs, edge_attr, wet, bcat)
    partials = _scatter_call(msg, dst, jnp.zeros((N, D), jnp.float32))
    outv = _final_call(x, partials, Wd, jnp.broadcast_to(bd, (1, D)))
    return outv[0, :1]


# speedup vs baseline: 3.7791x; 1.5270x over previous
import functools

import jax
import jax.numpy as jnp
from jax import lax
from jax.experimental import pallas as pl
from jax.experimental.pallas import tpu as pltpu
from jax.experimental.pallas import tpu_sc as plsc

N = 10000
E = 320000
D = 128
DE = 16
Z2 = 2 * D

NC = 2
NS = 16
NW = NC * NS
EPW = E // NW
CHUNK = 80
ROWS_PER_TILE = N // NS

_mesh = plsc.VectorSubcoreMesh(core_axis_name="c", subcore_axis_name="s")


GCH = 96
GNCH = EPW // GCH
GTAIL = EPW - GNCH * GCH


def _accum_rows(dst_buf, src_buf, nrows):
    def row(i, c):
        for j in range(Z2 // 16):
            plsc.addupdate(dst_buf.at[i, pl.ds(j * 16, 16)],
                           src_buf[i, pl.ds(j * 16, 16)])
        return c
    lax.fori_loop(0, nrows, row, 0)


@functools.partial(
    pl.kernel,
    out_type=jax.ShapeDtypeStruct((E, Z2), jnp.float32),
    mesh=_mesh,
    scratch_types=[
        pltpu.VMEM((EPW,), jnp.int32),
        pltpu.VMEM((EPW,), jnp.int32),
        pltpu.VMEM((GCH, Z2), jnp.float32),
        pltpu.VMEM((GCH, Z2), jnp.float32),
        pltpu.VMEM((GCH, Z2), jnp.float32),
        pltpu.VMEM((GCH, Z2), jnp.float32),
        pltpu.SemaphoreType.DMA,
        pltpu.SemaphoreType.DMA,
        pltpu.SemaphoreType.DMA,
        pltpu.SemaphoreType.DMA,
        pltpu.SemaphoreType.DMA,
        pltpu.SemaphoreType.DMA,
    ],
)
def _gather_call(td_hbm, ts_hbm, di_hbm, si_hbm, g_hbm,
                 idxd, idxs, d0, s0, d1, s1,
                 gd0, gs0, gd1, gs1, w0, w1):
    wid = lax.axis_index("s") * NC + lax.axis_index("c")
    base = wid * EPW

    dbuf = (d0, d1)
    sbuf = (s0, s1)
    gdsem = (gd0, gd1)
    gssem = (gs0, gs1)
    wsem = (w0, w1)

    def start_gather(b, off_local, n):
        pltpu.async_copy(td_hbm.at[idxd.at[pl.ds(off_local, n)]],
                         dbuf[b].at[pl.ds(0, n)], gdsem[b])
        pltpu.async_copy(ts_hbm.at[idxs.at[pl.ds(off_local, n)]],
                         sbuf[b].at[pl.ds(0, n)], gssem[b])

    def wait_gather(b, off_local, n):
        pltpu.make_async_copy(td_hbm.at[idxd.at[pl.ds(off_local, n)]],
                              dbuf[b].at[pl.ds(0, n)], gdsem[b]).wait()
        pltpu.make_async_copy(ts_hbm.at[idxs.at[pl.ds(off_local, n)]],
                              sbuf[b].at[pl.ds(0, n)], gssem[b]).wait()

    def start_wb(b, off_local, n):
        pltpu.async_copy(dbuf[b].at[pl.ds(0, n)],
                         g_hbm.at[pl.ds(base + off_local, n)], wsem[b])

    def wait_wb(b, off_local, n):
        pltpu.make_async_copy(dbuf[b].at[pl.ds(0, n)],
                              g_hbm.at[pl.ds(base + off_local, n)],
                              wsem[b]).wait()

    pltpu.sync_copy(di_hbm.at[pl.ds(base, EPW)], idxd)
    pltpu.sync_copy(si_hbm.at[pl.ds(base, EPW)], idxs)

    start_gather(0, 0, GCH)
    start_gather(1, GCH, GCH)

    def pair(k, carry):
        ce = 2 * k * GCH
        wait_gather(0, ce - 2 * GCH, GCH)
        _accum_rows(dbuf[0], sbuf[0], GCH)
        start_wb(0, ce - 2 * GCH, GCH)
        wait_gather(1, ce - GCH, GCH)
        _accum_rows(dbuf[1], sbuf[1], GCH)
        start_wb(1, ce - GCH, GCH)
        wait_wb(0, ce - 2 * GCH, GCH)
        start_gather(0, ce, GCH)
        wait_wb(1, ce - GCH, GCH)
        start_gather(1, ce + GCH, GCH)
        return carry

    lax.fori_loop(1, GNCH // 2, pair, 0)

    off0 = (GNCH - 2) * GCH
    off1 = (GNCH - 1) * GCH
    wait_gather(0, off0, GCH)
    _accum_rows(dbuf[0], sbuf[0], GCH)
    start_wb(0, off0, GCH)
    wait_gather(1, off1, GCH)
    _accum_rows(dbuf[1], sbuf[1], GCH)
    start_wb(1, off1, GCH)
    wait_wb(0, off0, GCH)
    toff = GNCH * GCH
    start_gather(0, toff, GTAIL)
    wait_gather(0, toff, GTAIL)
    _accum_rows(dbuf[0], sbuf[0], GTAIL)
    start_wb(0, toff, GTAIL)
    wait_wb(0, toff, GTAIL)
    wait_wb(1, off1, GCH)


@functools.partial(
    pl.kernel,
    out_type=jax.ShapeDtypeStruct((NC, N, D), jnp.float32),
    mesh=_mesh,
    scratch_types=[
        pltpu.VMEM((GCH,), jnp.int32),
        pltpu.VMEM((GCH,), jnp.int32),
        pltpu.VMEM((GTAIL,), jnp.int32),
        pltpu.VMEM((GCH, D), jnp.float32),
        pltpu.VMEM((GCH, D), jnp.float32),
        pltpu.VMEM_SHARED((N, D), jnp.float32),
        pltpu.SemaphoreType.DMA,
        pltpu.SemaphoreType.DMA,
    ],
)
def _scatter_call(msg_hbm, di_hbm, zeros_hbm, out_hbm,
                  i0, i1, i_t, m0, m1, agg_sh, r0sem, r1sem):
    cid = lax.axis_index("c")
    sid = lax.axis_index("s")
    wid = sid * NC + cid
    base = wid * EPW

    ibuf = (i0, i1)
    mbuf = (m0, m1)
    rsem = (r0sem, r1sem)

    @pl.when(sid < NS - 1)
    def _():
        pltpu.sync_copy(zeros_hbm.at[pl.ds(sid * 640, 640)],
                        agg_sh.at[pl.ds(sid * 640, 640)])

    @pl.when(sid == NS - 1)
    def _():
        pltpu.sync_copy(zeros_hbm.at[pl.ds((NS - 1) * 640, N - 640 * (NS - 1))],
                        agg_sh.at[pl.ds((NS - 1) * 640, N - 640 * (NS - 1))])

    plsc.subcore_barrier()

    def start_read(b, off_local, n):
        pltpu.async_copy(di_hbm.at[pl.ds(base + off_local, n)],
                         ibuf[b].at[pl.ds(0, n)], rsem[b])
        pltpu.async_copy(msg_hbm.at[pl.ds(base + off_local, n)],
                         mbuf[b].at[pl.ds(0, n)], rsem[b])

    def wait_read(b, off_local, n):
        pltpu.make_async_copy(di_hbm.at[pl.ds(base + off_local, n)],
                              ibuf[b].at[pl.ds(0, n)], rsem[b]).wait()
        pltpu.make_async_copy(msg_hbm.at[pl.ds(base + off_local, n)],
                              mbuf[b].at[pl.ds(0, n)], rsem[b]).wait()

    start_read(0, 0, GCH)
    start_read(1, GCH, GCH)

    def pair(k, carry):
        ce = 2 * k * GCH
        wait_read(0, ce - 2 * GCH, GCH)
        pltpu.sync_copy(m0, agg_sh.at[i0], add=True)
        start_read(0, ce, GCH)
        wait_read(1, ce - GCH, GCH)
        pltpu.sync_copy(m1, agg_sh.at[i1], add=True)
        start_read(1, ce + GCH, GCH)
        return carry

    lax.fori_loop(1, GNCH // 2, pair, 0)

    off0 = (GNCH - 2) * GCH
    off1 = (GNCH - 1) * GCH
    wait_read(0, off0, GCH)
    pltpu.sync_copy(m0, agg_sh.at[i0], add=True)
    wait_read(1, off1, GCH)
    pltpu.sync_copy(m1, agg_sh.at[i1], add=True)
    toff = GNCH * GCH
    pltpu.sync_copy(di_hbm.at[pl.ds(base + toff, GTAIL)], i_t)
    pltpu.sync_copy(msg_hbm.at[pl.ds(base + toff, GTAIL)], m0.at[pl.ds(0, GTAIL)])
    pltpu.sync_copy(m0.at[pl.ds(0, GTAIL)], agg_sh.at[i_t], add=True)

    plsc.subcore_barrier()

    def out_body(j, carry):
        r0 = sid * 640 + j * 80
        pltpu.sync_copy(agg_sh.at[pl.ds(r0, 80)], m0.at[pl.ds(0, 80)])
        pltpu.sync_copy(m0.at[pl.ds(0, 80)], out_hbm.at[cid, pl.ds(r0, 80)])
        return carry

    nchunks = jnp.where(sid == NS - 1, (N - 640 * (NS - 1)) // 80, 640 // 80)
    lax.fori_loop(0, nchunks, out_body, 0)


def _tables_body(x_ref, w1_ref, w2_ref, td_ref, ts_ref):
    xb = x_ref[...]
    td_ref[...] = jnp.dot(xb, w1_ref[...], preferred_element_type=jnp.float32)
    ts_ref[...] = jnp.dot(xb, w2_ref[...], preferred_element_type=jnp.float32)


BN_TAB = 2000
_tables_call = pl.pallas_call(
    _tables_body,
    grid=(N // BN_TAB,),
    in_specs=[
        pl.BlockSpec((BN_TAB, D), lambda i: (i, 0)),
        pl.BlockSpec((D, Z2), lambda i: (0, 0)),
        pl.BlockSpec((D, Z2), lambda i: (0, 0)),
    ],
    out_specs=[
        pl.BlockSpec((BN_TAB, Z2), lambda i: (i, 0)),
        pl.BlockSpec((BN_TAB, Z2), lambda i: (i, 0)),
    ],
    out_shape=[jax.ShapeDtypeStruct((N, Z2), jnp.float32)] * 2,
)


def _act_body(g_ref, ea_ref, wet_ref, bcat_ref, msg_ref):
    u = (g_ref[...]
         + jnp.dot(ea_ref[...], wet_ref[...], preferred_element_type=jnp.float32)
         + bcat_ref[...])
    gate = jax.nn.sigmoid(u[:, :D])
    filt = jax.nn.softplus(u[:, D:])
    msg_ref[...] = gate * filt


BE_ACT = 4000
_act_call = pl.pallas_call(
    _act_body,
    grid=(E // BE_ACT,),
    in_specs=[
        pl.BlockSpec((BE_ACT, Z2), lambda i: (i, 0)),
        pl.BlockSpec((BE_ACT, DE), lambda i: (i, 0)),
        pl.BlockSpec((DE, Z2), lambda i: (0, 0)),
        pl.BlockSpec((1, Z2), lambda i: (0, 0)),
    ],
    out_specs=pl.BlockSpec((BE_ACT, D), lambda i: (i, 0)),
    out_shape=jax.ShapeDtypeStruct((E, D), jnp.float32),
)


def _final_body(x_ref, p_ref, wd_ref, bd_ref, out_ref, acc_ref):
    i = pl.program_id(0)

    @pl.when(i == 0)
    def _():
        acc_ref[...] = jnp.zeros_like(acc_ref)

    s = jnp.maximum(x_ref[...] + p_ref[0] + p_ref[1], 0.0)
    acc_ref[...] += jnp.sum(s, axis=0, keepdims=True)

    @pl.when(i == pl.num_programs(0) - 1)
    def _():
        out_ref[...] = jnp.sum(acc_ref[...] * wd_ref[...]) + bd_ref[...]


BN_FIN = 1000
_final_call = pl.pallas_call(
    _final_body,
    grid=(N // BN_FIN,),
    in_specs=[
        pl.BlockSpec((BN_FIN, D), lambda i: (i, 0)),
        pl.BlockSpec((NC, BN_FIN, D), lambda i: (0, i, 0)),
        pl.BlockSpec((1, D), lambda i: (0, 0)),
        pl.BlockSpec((1, D), lambda i: (0, 0)),
    ],
    out_specs=pl.BlockSpec((1, D), lambda i: (0, 0)),
    out_shape=jax.ShapeDtypeStruct((1, D), jnp.float32),
    scratch_shapes=[pltpu.VMEM((1, D), jnp.float32)],
)


def kernel(x, edge_index, edge_attr, Wf, bf, Ws, bs, Wd, bd):
    ei = edge_index.astype(jnp.int32)
    src = ei[0]
    dst = ei[1]
    w1 = jnp.concatenate([Wf[:, :D], Ws[:, :D]], axis=0).T
    w2 = jnp.concatenate([Wf[:, D:2 * D], Ws[:, D:2 * D]], axis=0).T
    wet = jnp.concatenate([Wf[:, 2 * D:], Ws[:, 2 * D:]], axis=0).T
    bcat = jnp.concatenate([bf, bs]).reshape(1, Z2)

    td, ts = _tables_call(x, w1, w2)
    g = _gather_call(td, ts, dst, src)
    msg = _act_call(g, edge_attr, wet, bcat)
    partials = _scatter_call(msg, dst, jnp.zeros((N, D), jnp.float32))
    outv = _final_call(x, partials, Wd, jnp.broadcast_to(bd, (1, D)))
    return outv[0, :1]
